# Initial kernel scaffold; baseline (speedup 1.0000x reference)
import jax
import jax.numpy as jnp
from jax.experimental import pallas as pl


def kernel(self_x, neighbor_x, adj, self_weight, neighbor_weight, self_bias, neighbor_bias):
    raise NotImplementedError("write your pallas kernel here")


# speedup vs baseline: 1.1003x; 1.1003x over previous
import functools

import jax
import jax.numpy as jnp
from jax.experimental import pallas as pl
from jax.experimental.pallas import tpu as pltpu


def _fused_kernel(sx_ref, nx_ref, adj_ref, sw_ref, nw_ref, sb_ref, nb_ref,
                  out_ref):
    f32 = jnp.float32
    agg = jnp.dot(adj_ref[...], nx_ref[...], preferred_element_type=f32)
    nbr = jnp.dot(agg, nw_ref[...], preferred_element_type=f32) + nb_ref[...]
    slf = jnp.dot(sx_ref[...], sw_ref[...], preferred_element_type=f32) + sb_ref[...]
    out_ref[...] = jnp.maximum(jnp.concatenate([slf, nbr], axis=1), 0.0)


@functools.partial(jax.jit, static_argnames=("bm",))
def _run(self_x, neighbor_x, adj, self_weight, neighbor_weight,
         self_bias, neighbor_bias, bm):
    n, d_in = self_x.shape
    d_out = self_weight.shape[1]
    grid = (n // bm,)
    return pl.pallas_call(
        _fused_kernel,
        grid=grid,
        in_specs=[
            pl.BlockSpec((bm, d_in), lambda i: (i, 0)),
            pl.BlockSpec((n, d_in), lambda i: (0, 0)),
            pl.BlockSpec((bm, n), lambda i: (i, 0)),
            pl.BlockSpec((d_in, d_out), lambda i: (0, 0)),
            pl.BlockSpec((d_in, d_out), lambda i: (0, 0)),
            pl.BlockSpec((1, d_out), lambda i: (0, 0)),
            pl.BlockSpec((1, d_out), lambda i: (0, 0)),
        ],
        out_specs=pl.BlockSpec((bm, 2 * d_out), lambda i: (i, 0)),
        out_shape=jax.ShapeDtypeStruct((n, 2 * d_out), jnp.float32),
        compiler_params=pltpu.CompilerParams(
            dimension_semantics=("arbitrary",),
        ),
    )(self_x, neighbor_x, adj, self_weight, neighbor_weight,
      self_bias, neighbor_bias)


def kernel(self_x, neighbor_x, adj, self_weight, neighbor_weight,
           self_bias, neighbor_bias):
    n = adj.shape[0]
    bm = next(b for b in (400, 200, 100, 8, 1) if n % b == 0)
    sb = self_bias.reshape(1, -1)
    nb = neighbor_bias.reshape(1, -1)
    return _run(self_x, neighbor_x, adj, self_weight, neighbor_weight,
                sb, nb, bm)
